# Initial kernel scaffold; baseline (speedup 1.0000x reference)
import jax
import jax.numpy as jnp
from jax.experimental import pallas as pl


def kernel(input, mask_u):
    raise NotImplementedError("write your pallas kernel here")


# speedup vs baseline: 1.0007x; 1.0007x over previous
import jax
import jax.numpy as jnp
from jax.experimental import pallas as pl

_P = 0.5
_SCALE = 1.0 / (1.0 - _P)
_BM = 512


def _drop_kernel(x_ref, m_ref, o_ref):
    scale = jnp.where(m_ref[0, :] > _P, _SCALE, 0.0).astype(x_ref.dtype)
    o_ref[...] = x_ref[...] * scale[None, :]


def kernel(input, mask_u):
    m, n = input.shape
    mask2d = mask_u.reshape(1, n)
    return pl.pallas_call(
        _drop_kernel,
        grid=(m // _BM,),
        in_specs=[
            pl.BlockSpec((_BM, n), lambda i: (i, 0)),
            pl.BlockSpec((1, n), lambda i: (0, 0)),
        ],
        out_specs=pl.BlockSpec((_BM, n), lambda i: (i, 0)),
        out_shape=jax.ShapeDtypeStruct((m, n), input.dtype),
    )(input, mask2d)
